# baseline (speedup 1.0000x reference)
import jax
import jax.numpy as jnp
from jax import lax
from jax.experimental import pallas as pl
from jax.experimental.pallas import tpu as pltpu
from jax.experimental.pallas import tpu_sc as plsc

E = 256
N = 16384
NC = 2
NS = 16
NW = NC * NS
S = 4096
CPW = S // NW
CHUNK = 128
L = 16
G = CHUNK // L
TC_BLK = 1024
TC_GRID = (N - S) // TC_BLK


def _transform_slab(buf):
    zeros = tuple(jnp.zeros((L,), jnp.float32) for _ in range(G))

    @plsc.parallel_loop(1, E, carry=zeros, unroll=4)
    def accs(i, accs_in):
        return tuple(
            accs_in[g] + jnp.abs(buf[i, pl.ds(g * L, L)]) for g in range(G)
        )

    scales = []
    for g in range(G):
        sl = pl.ds(g * L, L)
        s1 = accs[g]
        x0 = buf[0, sl]
        lb = x0 - s1
        ub = x0 + s1
        crossing = (lb <= 0.0) & (ub >= 0.0)
        ub_le0 = ub <= 0.0
        alpha = 1.0 - lb
        scale = jnp.where(ub_le0, 0.0, jnp.where(crossing, alpha, 1.0))
        newc = alpha * x0 - alpha * lb * 0.5
        r0 = jnp.where(ub_le0, 0.0, jnp.where(crossing, newc, x0))
        buf[0, sl] = r0
        scales.append(scale)

    @plsc.parallel_loop(1, E, unroll=4)
    def _(i):
        for g in range(G):
            sl = pl.ds(g * L, L)
            buf[i, sl] = buf[i, sl] * scales[g]


def _tec_body(x_hbm, o_hbm, buf, sem_in, sem_out):
    wid = lax.axis_index("s") * NC + lax.axis_index("c")
    c0 = wid * CPW
    pltpu.async_copy(x_hbm.at[:, pl.ds(c0, CHUNK)], buf, sem_in).wait()
    _transform_slab(buf)
    pltpu.async_copy(buf, o_hbm.at[:, pl.ds(c0, CHUNK)], sem_out).wait()


def _sc_part(x):
    run = pl.kernel(
        _tec_body,
        out_type=jax.ShapeDtypeStruct((E, S), jnp.float32),
        mesh=plsc.VectorSubcoreMesh(core_axis_name="c", subcore_axis_name="s"),
        scratch_types=[
            pltpu.VMEM((E, CHUNK), jnp.float32),
            pltpu.SemaphoreType.DMA,
            pltpu.SemaphoreType.DMA,
        ],
    )
    return run(x)


def _tc_block(x_ref, o_ref):
    xb = x_ref[...]
    x0 = xb[0, :]
    s1 = jnp.sum(jnp.abs(xb), axis=0) - jnp.abs(x0)
    lb = x0 - s1
    ub = x0 + s1
    crossing = (lb <= 0.0) & (ub >= 0.0)
    ub_le0 = ub <= 0.0
    alpha = 1.0 - lb
    scale = jnp.where(ub_le0, 0.0, jnp.where(crossing, alpha, 1.0))
    newc = alpha * x0 - alpha * lb * 0.5
    r0 = jnp.where(ub_le0, 0.0, jnp.where(crossing, newc, x0))
    o_ref[...] = xb * scale[None, :]
    o_ref[0, :] = r0


def _tc_part(x):
    return pl.pallas_call(
        _tc_block,
        grid=(TC_GRID,),
        in_specs=[
            pl.BlockSpec((E, TC_BLK), lambda j: (0, j + S // TC_BLK)),
        ],
        out_specs=pl.BlockSpec((E, TC_BLK), lambda j: (0, j + S // TC_BLK)),
        out_shape=jax.ShapeDtypeStruct((E, N), jnp.float32),
        compiler_params=pltpu.CompilerParams(
            dimension_semantics=("parallel",),
        ),
    )(x)


def kernel(x):
    tc_out = _tc_part(x)
    sc_out = _sc_part(x)
    return lax.dynamic_update_slice(tc_out, sc_out, (0, 0))


# speedup vs baseline: 1.0006x; 1.0006x over previous
import jax
import jax.numpy as jnp
from jax import lax
from jax.experimental import pallas as pl
from jax.experimental.pallas import tpu as pltpu
from jax.experimental.pallas import tpu_sc as plsc

E = 256
N = 16384
NC = 2
NS = 16
NW = NC * NS
CPW = N // NW
CHUNK = 128
NCHUNKS = CPW // CHUNK
NBUF = 3
L = 16
G = CHUNK // L


def _transform_slab(buf):
    def red(i, accs):
        return tuple(
            accs[g] + jnp.abs(buf[i, pl.ds(g * L, L)]) for g in range(G)
        )

    zeros = tuple(jnp.zeros((L,), jnp.float32) for _ in range(G))
    accs = lax.fori_loop(1, E, red, zeros)

    scales = []
    for g in range(G):
        sl = pl.ds(g * L, L)
        s1 = accs[g]
        x0 = buf[0, sl]
        lb = x0 - s1
        ub = x0 + s1
        crossing = (lb <= 0.0) & (ub >= 0.0)
        ub_le0 = ub <= 0.0
        alpha = 1.0 - lb
        scale = jnp.where(ub_le0, 0.0, jnp.where(crossing, alpha, 1.0))
        newc = alpha * x0 - alpha * lb * 0.5
        r0 = jnp.where(ub_le0, 0.0, jnp.where(crossing, newc, x0))
        buf[0, sl] = r0
        scales.append(scale)

    def scl(i, carry):
        for g in range(G):
            sl = pl.ds(g * L, L)
            buf[i, sl] = buf[i, sl] * scales[g]
        return carry

    lax.fori_loop(1, E, scl, 0)


def _tec_body(x_hbm, o_hbm, bufs, sems_in, sems_out):
    wid = lax.axis_index("s") * NC + lax.axis_index("c")
    base = wid * CPW

    def col0(k):
        return base + k * CHUNK

    h_in = [None] * NCHUNKS
    h_out = [None] * NCHUNKS
    h_in[0] = pltpu.async_copy(
        x_hbm.at[:, pl.ds(col0(0), CHUNK)], bufs[0], sems_in[0]
    )
    for k in range(NCHUNKS):
        b = k % NBUF
        nxt = (k + 1) % NBUF
        if k + 1 < NCHUNKS:
            if k + 1 - NBUF >= 0:
                h_out[k + 1 - NBUF].wait()
            h_in[k + 1] = pltpu.async_copy(
                x_hbm.at[:, pl.ds(col0(k + 1), CHUNK)], bufs[nxt], sems_in[nxt]
            )
        h_in[k].wait()
        _transform_slab(bufs[b])
        h_out[k] = pltpu.async_copy(
            bufs[b], o_hbm.at[:, pl.ds(col0(k), CHUNK)], sems_out[b]
        )
    for k in range(max(0, NCHUNKS - NBUF + 1), NCHUNKS):
        h_out[k].wait()


def kernel(x):
    run = pl.kernel(
        _tec_body,
        out_type=jax.ShapeDtypeStruct((E, N), jnp.float32),
        mesh=plsc.VectorSubcoreMesh(core_axis_name="c", subcore_axis_name="s"),
        scratch_types=[
            [pltpu.VMEM((E, CHUNK), jnp.float32) for _ in range(NBUF)],
            [pltpu.SemaphoreType.DMA for _ in range(NBUF)],
            [pltpu.SemaphoreType.DMA for _ in range(NBUF)],
        ],
    )
    return run(x)


# speedup vs baseline: 1.0038x; 1.0032x over previous
import jax
import jax.numpy as jnp
from jax import lax
from jax.experimental import pallas as pl
from jax.experimental.pallas import tpu as pltpu
from jax.experimental.pallas import tpu_sc as plsc

E = 256
H = E // 2
N = 16384
NC = 2
NS = 16
NW = NC * NS
CPW = N // NW
CHUNK = 128
NCHUNKS = CPW // CHUNK
NBUF = 3
L = 16
G = CHUNK // L


def _reduce_half(buf, lo, hi, accs):
    def red(i, a):
        return tuple(
            a[g] + jnp.abs(buf[i, pl.ds(g * L, L)]) for g in range(G)
        )

    return lax.fori_loop(lo, hi, red, accs)


def _scale_half(buf, lo, hi, scales):
    def scl(i, carry):
        for g in range(G):
            sl = pl.ds(g * L, L)
            buf[i, sl] = buf[i, sl] * scales[g]
        return carry

    lax.fori_loop(lo, hi, scl, 0)


def _tec_body(x_hbm, o_hbm, bufs, sems_ia, sems_ib, sems_oa, sems_ob):
    wid = lax.axis_index("s") * NC + lax.axis_index("c")
    base = wid * CPW

    def gather(k, b):
        c0 = base + k * CHUNK
        ha = pltpu.async_copy(
            x_hbm.at[pl.ds(0, H), pl.ds(c0, CHUNK)],
            bufs[b].at[pl.ds(0, H), :],
            sems_ia[b],
        )
        hb = pltpu.async_copy(
            x_hbm.at[pl.ds(H, H), pl.ds(c0, CHUNK)],
            bufs[b].at[pl.ds(H, H), :],
            sems_ib[b],
        )
        return ha, hb

    h_in = [None] * NCHUNKS
    h_out = [None] * NCHUNKS
    h_in[0] = gather(0, 0)
    for k in range(NCHUNKS):
        b = k % NBUF
        nxt = (k + 1) % NBUF
        if k + 1 < NCHUNKS:
            if k + 1 - NBUF >= 0:
                h_out[k + 1 - NBUF][0].wait()
                h_out[k + 1 - NBUF][1].wait()
            h_in[k + 1] = gather(k + 1, nxt)
        buf = bufs[b]
        c0 = base + k * CHUNK

        h_in[k][0].wait()
        zeros = tuple(jnp.zeros((L,), jnp.float32) for _ in range(G))
        accs = _reduce_half(buf, 1, H, zeros)
        h_in[k][1].wait()
        accs = _reduce_half(buf, H, E, accs)

        scales = []
        row0 = []
        for g in range(G):
            sl = pl.ds(g * L, L)
            s1 = accs[g]
            x0 = buf[0, sl]
            lb = x0 - s1
            ub = x0 + s1
            crossing = (lb <= 0.0) & (ub >= 0.0)
            ub_le0 = ub <= 0.0
            alpha = 1.0 - lb
            scale = jnp.where(ub_le0, 0.0, jnp.where(crossing, alpha, 1.0))
            newc = alpha * x0 - alpha * lb * 0.5
            r0 = jnp.where(ub_le0, 0.0, jnp.where(crossing, newc, x0))
            scales.append(scale)
            row0.append(r0)

        _scale_half(buf, 1, H, scales)
        for g in range(G):
            buf[0, pl.ds(g * L, L)] = row0[g]
        oa = pltpu.async_copy(
            buf.at[pl.ds(0, H), :],
            o_hbm.at[pl.ds(0, H), pl.ds(c0, CHUNK)],
            sems_oa[b],
        )
        _scale_half(buf, H, E, scales)
        ob = pltpu.async_copy(
            buf.at[pl.ds(H, H), :],
            o_hbm.at[pl.ds(H, H), pl.ds(c0, CHUNK)],
            sems_ob[b],
        )
        h_out[k] = (oa, ob)
    for k in range(max(0, NCHUNKS - NBUF + 1), NCHUNKS):
        h_out[k][0].wait()
        h_out[k][1].wait()


def kernel(x):
    run = pl.kernel(
        _tec_body,
        out_type=jax.ShapeDtypeStruct((E, N), jnp.float32),
        mesh=plsc.VectorSubcoreMesh(core_axis_name="c", subcore_axis_name="s"),
        scratch_types=[
            [pltpu.VMEM((E, CHUNK), jnp.float32) for _ in range(NBUF)],
            [pltpu.SemaphoreType.DMA for _ in range(NBUF)],
            [pltpu.SemaphoreType.DMA for _ in range(NBUF)],
            [pltpu.SemaphoreType.DMA for _ in range(NBUF)],
            [pltpu.SemaphoreType.DMA for _ in range(NBUF)],
        ],
    )
    return run(x)
